# baseline (speedup 1.0000x reference)
import functools

import jax
import jax.numpy as jnp
from jax import lax
from jax.experimental import pallas as pl
from jax.experimental.pallas import tpu as pltpu
from jax.experimental.pallas import tpu_sc as plsc

_LANE = 128
_VL = 16


def kernel(x, embedding):
    B, S = x.shape
    V, D = embedding.shape
    N = B * S
    NC, NS = 2, 16
    NW = NC * NS
    rows = N // _LANE
    rows_w = rows // NW
    R = 16
    n_g = rows_w // R

    mesh = plsc.VectorSubcoreMesh(core_axis_name="c", subcore_axis_name="s")

    @functools.partial(
        pl.kernel,
        mesh=mesh,
        out_type=jax.ShapeDtypeStruct((2 * rows, _LANE), jnp.float32),
        scratch_types=[
            pltpu.VMEM((R, _LANE), jnp.int32),
            pltpu.VMEM((2 * R, _LANE), jnp.int32),
            pltpu.VMEM((2 * R, _LANE), jnp.float32),
            pltpu.VMEM((2 * R, _LANE), jnp.float32),
            pltpu.SemaphoreType.DMA,
        ],
    )
    def gather_k(idx_hbm, tab_hbm, out_hbm, idx_v, widx_v, cs_v, out_v, sem):
        wid = lax.axis_index("s") * NC + lax.axis_index("c")
        base = wid * rows_w
        lane = lax.iota(jnp.int32, _VL)
        perm_lo = lax.shift_right_logical(lane, 1)
        perm_hi = perm_lo + 8
        odd = lax.bitwise_and(lane, 1) == 1
        _dnums = lax.GatherDimensionNumbers(
            offset_dims=(), collapsed_slice_dims=(0,), start_index_map=(0,))

        def lane_gather(vec, perm):
            return lax.gather(
                vec, perm.reshape(_VL, 1), dimension_numbers=_dnums,
                slice_sizes=(1,),
                mode=lax.GatherScatterMode.PROMISE_IN_BOUNDS)

        def widx_row(j):
            for t in range(8):
                v = idx_v[j, pl.ds(_VL * t, _VL)]
                e = v * 2
                widx_v[j, pl.ds(_VL * t, _VL)] = e
                widx_v[R + j, pl.ds(_VL * t, _VL)] = e + 1

        def interleave_row(j):
            for u in range(8):
                c = cs_v[j, pl.ds(_VL * u, _VL)]
                s = cs_v[R + j, pl.ds(_VL * u, _VL)]
                lo = jnp.where(odd, lane_gather(s, perm_lo),
                               lane_gather(c, perm_lo))
                hi = jnp.where(odd, lane_gather(s, perm_hi),
                               lane_gather(c, perm_hi))
                r = 2 * j + u // 4
                col = (32 * u) % _LANE
                out_v[r, pl.ds(col, _VL)] = lo
                out_v[r, pl.ds(col + _VL, _VL)] = hi

        def group(g, carry):
            off = base + g * R
            pltpu.sync_copy(idx_hbm.at[pl.ds(off, R)], idx_v)
            for j in range(R):
                widx_row(j)
            cps = [pltpu.async_copy(tab_hbm.at[widx_v.at[j]], cs_v.at[j], sem)
                   for j in range(2 * R)]
            for c in cps:
                c.wait()
            for j in range(R):
                interleave_row(j)
            pltpu.sync_copy(out_v, out_hbm.at[pl.ds(2 * off, 2 * R)])
            return carry

        lax.fori_loop(0, n_g, group, 0)

    out = gather_k(x.reshape(rows, _LANE), embedding.reshape(2 * V))
    return out.reshape(B, S, D)


# speedup vs baseline: 8.4200x; 8.4200x over previous
import functools

import jax
import jax.numpy as jnp
from jax import lax
from jax.experimental import pallas as pl
from jax.experimental.pallas import tpu as pltpu
from jax.experimental.pallas import tpu_sc as plsc

_LANE = 128


def kernel(x, embedding):
    B, S = x.shape
    V, D = embedding.shape
    N = B * S
    NC, NS = 2, 16
    NW = NC * NS
    rows = N // _LANE
    rows_w = rows // NW
    R = 16
    n_g = rows_w // R

    mesh = plsc.VectorSubcoreMesh(core_axis_name="c", subcore_axis_name="s")

    @functools.partial(
        pl.kernel,
        mesh=mesh,
        out_type=(
            jax.ShapeDtypeStruct((rows, _LANE), jnp.float32),
            jax.ShapeDtypeStruct((rows, _LANE), jnp.float32),
        ),
        scratch_types=[
            pltpu.VMEM((R, _LANE), jnp.int32),
            pltpu.VMEM((R, _LANE), jnp.int32),
            pltpu.VMEM((R, _LANE), jnp.float32),
            pltpu.VMEM((R, _LANE), jnp.float32),
            pltpu.VMEM((R, _LANE), jnp.float32),
            pltpu.VMEM((R, _LANE), jnp.float32),
            pltpu.SemaphoreType.DMA,
            pltpu.SemaphoreType.DMA,
            pltpu.SemaphoreType.DMA,
            pltpu.SemaphoreType.DMA,
            pltpu.SemaphoreType.DMA,
            pltpu.SemaphoreType.DMA,
        ],
    )
    def gather_k(idx_hbm, cos_hbm, sin_hbm, cos_out, sin_out,
                 i0, i1, c0, c1, s0, s1,
                 gs0, gs1, ws0, ws1, is0, is1):
        ibuf = (i0, i1)
        cbuf = (c0, c1)
        sbuf = (s0, s1)
        gsem = (gs0, gs1)
        wsem = (ws0, ws1)
        isem = (is0, is1)
        wid = lax.axis_index("s") * NC + lax.axis_index("c")
        base = wid * rows_w

        def stage(g, b):
            pltpu.async_copy(idx_hbm.at[pl.ds(base + g * R, R)],
                             ibuf[b], isem[b])

        def wait_idx(b):
            pltpu.make_async_copy(idx_hbm.at[pl.ds(0, R)],
                                  ibuf[b], isem[b]).wait()

        def fire(b):
            for j in range(R):
                pltpu.async_copy(cos_hbm.at[ibuf[b].at[j]],
                                 cbuf[b].at[j], gsem[b])
                pltpu.async_copy(sin_hbm.at[ibuf[b].at[j]],
                                 sbuf[b].at[j], gsem[b])

        def wait_gather(b):
            pltpu.make_async_copy(cos_out.at[pl.ds(0, R)],
                                  cbuf[b], gsem[b]).wait()
            pltpu.make_async_copy(sin_out.at[pl.ds(0, R)],
                                  sbuf[b], gsem[b]).wait()

        def writeback(g, b):
            off = base + g * R
            pltpu.async_copy(cbuf[b], cos_out.at[pl.ds(off, R)], wsem[b])
            pltpu.async_copy(sbuf[b], sin_out.at[pl.ds(off, R)], wsem[b])

        def wait_wb(b):
            pltpu.make_async_copy(cbuf[b], cos_out.at[pl.ds(0, R)],
                                  wsem[b]).wait()
            pltpu.make_async_copy(sbuf[b], sin_out.at[pl.ds(0, R)],
                                  wsem[b]).wait()

        def step(k, b, do_wait_wb, do_stage):
            nb = 1 - b
            if do_wait_wb:
                wait_wb(nb)
            wait_idx(nb)
            wait_gather(b)
            fire(nb)
            writeback(k, b)
            if do_stage:
                stage_guarded(k + 2, b)

        def stage_guarded(g, b):
            @pl.when(g < n_g)
            def _():
                stage(g, b)

        stage(0, 0)
        wait_idx(0)
        fire(0)
        stage(1, 1)
        step(0, 0, do_wait_wb=False, do_stage=True)

        def body(k2, carry):
            step(2 * k2 + 1, 1, do_wait_wb=True, do_stage=True)
            step(2 * k2 + 2, 0, do_wait_wb=True, do_stage=True)
            return carry

        lax.fori_loop(0, (n_g - 2) // 2, body, 0)

        wait_gather(1)
        writeback(n_g - 1, 1)
        wait_wb(0)
        wait_wb(1)

    cos_t = jax.lax.slice_in_dim(embedding, 0, 1, axis=1).reshape(V)
    sin_t = jax.lax.slice_in_dim(embedding, 1, 2, axis=1).reshape(V)
    cos_p, sin_p = gather_k(x.reshape(rows, _LANE), cos_t, sin_t)
    out = jnp.stack([cos_p.reshape(N), sin_p.reshape(N)], axis=-1)
    return out.reshape(B, S, D)


# speedup vs baseline: 8.6778x; 1.0306x over previous
import functools

import jax
import jax.numpy as jnp
from jax import lax
from jax.experimental import pallas as pl
from jax.experimental.pallas import tpu as pltpu
from jax.experimental.pallas import tpu_sc as plsc

_LANE = 128


def kernel(x, embedding):
    B, S = x.shape
    V, D = embedding.shape
    N = B * S
    NC, NS = 2, 16
    NW = NC * NS
    rows = N // _LANE
    rows_w = rows // NW
    R = 32
    n_g = rows_w // R

    mesh = plsc.VectorSubcoreMesh(core_axis_name="c", subcore_axis_name="s")

    @functools.partial(
        pl.kernel,
        mesh=mesh,
        out_type=(
            jax.ShapeDtypeStruct((rows, _LANE), jnp.float32),
            jax.ShapeDtypeStruct((rows, _LANE), jnp.float32),
        ),
        scratch_types=[
            pltpu.VMEM((R, _LANE), jnp.int32),
            pltpu.VMEM((R, _LANE), jnp.int32),
            pltpu.VMEM((R, _LANE), jnp.float32),
            pltpu.VMEM((R, _LANE), jnp.float32),
            pltpu.VMEM((R, _LANE), jnp.float32),
            pltpu.VMEM((R, _LANE), jnp.float32),
            pltpu.SemaphoreType.DMA,
            pltpu.SemaphoreType.DMA,
            pltpu.SemaphoreType.DMA,
            pltpu.SemaphoreType.DMA,
            pltpu.SemaphoreType.DMA,
            pltpu.SemaphoreType.DMA,
        ],
    )
    def gather_k(idx_hbm, cos_hbm, sin_hbm, cos_out, sin_out,
                 i0, i1, c0, c1, s0, s1,
                 gs0, gs1, ws0, ws1, is0, is1):
        ibuf = (i0, i1)
        cbuf = (c0, c1)
        sbuf = (s0, s1)
        gsem = (gs0, gs1)
        wsem = (ws0, ws1)
        isem = (is0, is1)
        wid = lax.axis_index("s") * NC + lax.axis_index("c")
        base = wid * rows_w

        def stage(g, b):
            pltpu.async_copy(idx_hbm.at[pl.ds(base + g * R, R)],
                             ibuf[b], isem[b])

        def wait_idx(b):
            pltpu.make_async_copy(idx_hbm.at[pl.ds(0, R)],
                                  ibuf[b], isem[b]).wait()

        def fire(b):
            for j in range(R):
                pltpu.async_copy(cos_hbm.at[ibuf[b].at[j]],
                                 cbuf[b].at[j], gsem[b])
                pltpu.async_copy(sin_hbm.at[ibuf[b].at[j]],
                                 sbuf[b].at[j], gsem[b])

        def wait_gather(b):
            pltpu.make_async_copy(cos_out.at[pl.ds(0, R)],
                                  cbuf[b], gsem[b]).wait()
            pltpu.make_async_copy(sin_out.at[pl.ds(0, R)],
                                  sbuf[b], gsem[b]).wait()

        def writeback(g, b):
            off = base + g * R
            pltpu.async_copy(cbuf[b], cos_out.at[pl.ds(off, R)], wsem[b])
            pltpu.async_copy(sbuf[b], sin_out.at[pl.ds(off, R)], wsem[b])

        def wait_wb(b):
            pltpu.make_async_copy(cbuf[b], cos_out.at[pl.ds(0, R)],
                                  wsem[b]).wait()
            pltpu.make_async_copy(sbuf[b], sin_out.at[pl.ds(0, R)],
                                  wsem[b]).wait()

        def step(k, b, do_wait_wb, do_stage):
            nb = 1 - b
            if do_wait_wb:
                wait_wb(nb)
            wait_idx(nb)
            wait_gather(b)
            fire(nb)
            writeback(k, b)
            if do_stage:
                stage_guarded(k + 2, b)

        def stage_guarded(g, b):
            @pl.when(g < n_g)
            def _():
                stage(g, b)

        stage(0, 0)
        wait_idx(0)
        fire(0)
        stage(1, 1)
        step(0, 0, do_wait_wb=False, do_stage=True)

        def body(k2, carry):
            step(2 * k2 + 1, 1, do_wait_wb=True, do_stage=True)
            step(2 * k2 + 2, 0, do_wait_wb=True, do_stage=True)
            return carry

        lax.fori_loop(0, (n_g - 2) // 2, body, 0)

        if (n_g - 2) % 2:
            step(n_g - 2, (n_g - 2) % 2, do_wait_wb=True, do_stage=True)

        b_last = (n_g - 1) % 2
        wait_gather(b_last)
        writeback(n_g - 1, b_last)
        wait_wb(1 - b_last)
        wait_wb(b_last)

    cos_t = jax.lax.slice_in_dim(embedding, 0, 1, axis=1).reshape(V)
    sin_t = jax.lax.slice_in_dim(embedding, 1, 2, axis=1).reshape(V)
    cos_p, sin_p = gather_k(x.reshape(rows, _LANE), cos_t, sin_t)
    out = jnp.stack([cos_p.reshape(N), sin_p.reshape(N)], axis=-1)
    return out.reshape(B, S, D)


# speedup vs baseline: 8.9734x; 1.0341x over previous
import functools

import jax
import jax.numpy as jnp
from jax import lax
from jax.experimental import pallas as pl
from jax.experimental.pallas import tpu as pltpu
from jax.experimental.pallas import tpu_sc as plsc

_LANE = 128


def kernel(x, embedding):
    B, S = x.shape
    V, D = embedding.shape
    N = B * S
    NC, NS = 2, 16
    NW = NC * NS
    rows = N // _LANE
    rows_w = rows // NW
    R = 32
    n_g = rows_w // R

    mesh = plsc.VectorSubcoreMesh(core_axis_name="c", subcore_axis_name="s")

    @functools.partial(
        pl.kernel,
        mesh=mesh,
        out_type=(
            jax.ShapeDtypeStruct((rows, _LANE), jnp.float32),
            jax.ShapeDtypeStruct((rows, _LANE), jnp.float32),
        ),
        scratch_types=[
            pltpu.VMEM((R, _LANE), jnp.int32),
            pltpu.VMEM((R, _LANE), jnp.int32),
            pltpu.VMEM((R, _LANE), jnp.float32),
            pltpu.VMEM((R, _LANE), jnp.float32),
            pltpu.VMEM((R, _LANE), jnp.float32),
            pltpu.VMEM((R, _LANE), jnp.float32),
            pltpu.SemaphoreType.DMA,
            pltpu.SemaphoreType.DMA,
            pltpu.SemaphoreType.DMA,
            pltpu.SemaphoreType.DMA,
            pltpu.SemaphoreType.DMA,
            pltpu.SemaphoreType.DMA,
        ],
    )
    def gather_k(idx_hbm, cos_hbm, sin_hbm, cos_out, sin_out,
                 i0, i1, c0, c1, s0, s1,
                 gs0, gs1, ws0, ws1, is0, is1):
        ibuf = (i0, i1)
        cbuf = (c0, c1)
        sbuf = (s0, s1)
        gsem = (gs0, gs1)
        wsem = (ws0, ws1)
        isem = (is0, is1)
        wid = lax.axis_index("s") * NC + lax.axis_index("c")
        base = wid * rows_w

        def stage(g, b):
            pltpu.async_copy(idx_hbm.at[pl.ds(base + g * R, R)],
                             ibuf[b], isem[b])

        def wait_idx(b):
            pltpu.make_async_copy(idx_hbm.at[pl.ds(0, R)],
                                  ibuf[b], isem[b]).wait()

        def fire(b):
            for j in range(R):
                pltpu.async_copy(cos_hbm.at[ibuf[b].at[j]],
                                 cbuf[b].at[j], gsem[b])
                pltpu.async_copy(sin_hbm.at[ibuf[b].at[j]],
                                 sbuf[b].at[j], gsem[b])

        def wait_gather(b):
            pltpu.make_async_copy(cos_out.at[pl.ds(0, R)],
                                  cbuf[b], gsem[b]).wait()
            pltpu.make_async_copy(sin_out.at[pl.ds(0, R)],
                                  sbuf[b], gsem[b]).wait()

        def writeback(g, b):
            off = base + g * R
            pltpu.async_copy(cbuf[b], cos_out.at[pl.ds(off, R)], wsem[b])
            pltpu.async_copy(sbuf[b], sin_out.at[pl.ds(off, R)], wsem[b])

        def wait_wb(b):
            pltpu.make_async_copy(cbuf[b], cos_out.at[pl.ds(0, R)],
                                  wsem[b]).wait()
            pltpu.make_async_copy(sbuf[b], sin_out.at[pl.ds(0, R)],
                                  wsem[b]).wait()

        def step(k, b, do_wait_wb, do_stage):
            nb = 1 - b
            if do_wait_wb:
                wait_wb(nb)
            wait_idx(nb)
            fire(nb)
            wait_gather(b)
            writeback(k, b)
            if do_stage:
                stage_guarded(k + 2, b)

        def stage_guarded(g, b):
            @pl.when(g < n_g)
            def _():
                stage(g, b)

        stage(0, 0)
        wait_idx(0)
        fire(0)
        stage(1, 1)
        step(0, 0, do_wait_wb=False, do_stage=True)

        def body(k2, carry):
            step(2 * k2 + 1, 1, do_wait_wb=True, do_stage=True)
            step(2 * k2 + 2, 0, do_wait_wb=True, do_stage=True)
            return carry

        lax.fori_loop(0, (n_g - 2) // 2, body, 0)

        if (n_g - 2) % 2:
            step(n_g - 2, (n_g - 2) % 2, do_wait_wb=True, do_stage=True)

        b_last = (n_g - 1) % 2
        wait_gather(b_last)
        writeback(n_g - 1, b_last)
        wait_wb(1 - b_last)
        wait_wb(b_last)

    cos_t = jax.lax.slice_in_dim(embedding, 0, 1, axis=1).reshape(V)
    sin_t = jax.lax.slice_in_dim(embedding, 1, 2, axis=1).reshape(V)
    cos_p, sin_p = gather_k(x.reshape(rows, _LANE), cos_t, sin_t)
    out = jnp.stack([cos_p.reshape(N), sin_p.reshape(N)], axis=-1)
    return out.reshape(B, S, D)
